# baseline (speedup 1.0000x reference)
import functools
import jax
import jax.numpy as jnp
from jax import lax
from jax.experimental import pallas as pl
from jax.experimental.pallas import tpu as pltpu
from jax.experimental.pallas import tpu_sc as plsc

N = 10000
E = 160000
D = 256
H = 128
NP = 10240
RT = 512
NRT = NP // RT
K = 128
NCHUNK = E // K
NTILES = 16
NW = 32
EPW = E // NW

_mesh = lambda: plsc.VectorSubcoreMesh(core_axis_name="c", subcore_axis_name="s")


def _deg_body(dst_hbm, out_hbm, dbuf, acc):
    c = lax.axis_index("c")
    s = lax.axis_index("s")
    w = s * 2 + c

    def zero(i, _):
        acc[pl.ds(i * 16, 16)] = jnp.zeros((16,), jnp.float32)
        return 0
    lax.fori_loop(0, NP // 16, zero, 0)

    dbuf[pl.ds(EPW - 16, 16)] = jnp.zeros((16,), jnp.int32)
    dbuf[pl.ds(EPW, 16)] = jnp.zeros((16,), jnp.int32)
    pltpu.sync_copy(dst_hbm.at[pl.ds(w * EPW, EPW)], dbuf.at[pl.ds(0, EPW)])

    ones = jnp.full((16,), 1.0, jnp.float32)

    def body(i, _):
        idx = dbuf[pl.ds(i * 16, 16)]
        plsc.addupdate_scatter(acc, [idx], ones)
        return 0
    lax.fori_loop(0, EPW // 16, body, 0)
    idx = dbuf[pl.ds((EPW // 16) * 16, 16)]
    mask = lax.broadcasted_iota(jnp.int32, (16,), 0) < (EPW % 16)
    plsc.addupdate_scatter(acc, [idx], ones, mask=mask)

    pltpu.sync_copy(acc, out_hbm.at[w])


@functools.partial(
    pl.kernel,
    out_type=jax.ShapeDtypeStruct((NW, NP), jnp.float32),
    mesh=_mesh(),
    scratch_types=[
        pltpu.VMEM((EPW + 16,), jnp.int32),
        pltpu.VMEM((NP,), jnp.float32),
    ],
    compiler_params=pltpu.CompilerParams(needs_layout_passes=False),
)
def _deg_kernel(dst_hbm, out_hbm, dbuf, acc):
    _deg_body(dst_hbm, out_hbm, dbuf, acc)


CH = 128
EP = 163840
NCH = EP // CH
CPT = NCH // NTILES
ACCR = NP + CH
IPT = NP // NTILES


def _agg_init(tbl, acc, s, rows2):
    for k in range(IPT // CH):
        r = s * IPT + k * CH
        pltpu.sync_copy(tbl.at[pl.ds(r, CH)], rows2)
        pltpu.sync_copy(rows2, acc.at[pl.ds(r, CH)])


GRP = 8
GPT = CPT // GRP


def _agg_edges(tbl, acc, c0, src2d, dst2d, si, di, rows):
    def body(g, _):
        pltpu.sync_copy(src2d.at[pl.ds(c0 + g * GRP, GRP)], si)
        pltpu.sync_copy(dst2d.at[pl.ds(c0 + g * GRP, GRP)], di)
        for k in range(GRP):
            pltpu.sync_copy(tbl.at[si.at[k]], rows)
            pltpu.sync_copy(rows, acc.at[di.at[k]], add=True)
        return 0
    lax.fori_loop(0, GPT, body, 0)


def _agg_writeout(out, acc, s, rows2):
    for k in range(IPT // CH):
        r = s * IPT + k * CH
        pltpu.sync_copy(acc.at[pl.ds(r, CH)], rows2)
        pltpu.sync_copy(rows2, out.at[pl.ds(r, CH)])


@functools.partial(
    pl.kernel,
    out_type=(
        jax.ShapeDtypeStruct((NP, H), jnp.float32),
        jax.ShapeDtypeStruct((NP, H), jnp.float32),
    ),
    mesh=_mesh(),
    scratch_types=[
        pltpu.VMEM((GRP, CH), jnp.int32),
        pltpu.VMEM((GRP, CH), jnp.int32),
        pltpu.VMEM((CH, H), jnp.float32),
        pltpu.VMEM_SHARED((ACCR, H), jnp.float32),
    ],
    compiler_params=pltpu.CompilerParams(needs_layout_passes=False),
)
def _agg_kernel(h0, h1, src2d, dst2d, o0, o1,
                si, di, rows, acc):
    c = lax.axis_index("c")
    s = lax.axis_index("s")

    @pl.when(c == 0)
    def _():
        _agg_init(h0, acc, s, rows)

    @pl.when(c == 1)
    def _():
        _agg_init(h1, acc, s, rows)

    plsc.subcore_barrier()

    @pl.when(c == 0)
    def _():
        _agg_edges(h0, acc, s * CPT, src2d, dst2d, si, di, rows)

    @pl.when(c == 1)
    def _():
        _agg_edges(h1, acc, s * CPT, src2d, dst2d, si, di, rows)

    plsc.subcore_barrier()

    @pl.when(c == 0)
    def _():
        _agg_writeout(o0, acc, s, rows)

    @pl.when(c == 1)
    def _():
        _agg_writeout(o1, acc, s, rows)


def _tc1_body(x_ref, w_ref, degp_ref, h0_ref, h1_ref, dis_ref):
    j = pl.program_id(0)
    deg = jnp.sum(degp_ref[...], axis=0) + 1.0
    dis = lax.rsqrt(deg)
    mm = jnp.dot(x_ref[...], w_ref[...], preferred_element_type=jnp.float32)
    row = j * RT + lax.broadcasted_iota(jnp.int32, (RT, 1), 0)
    hp = jnp.where(row < N, mm * dis[:, None], 0.0)
    h0_ref[...] = hp[:, :H]
    h1_ref[...] = hp[:, H:]
    dis_ref[...] = dis[:, None]


def _tc1(x, W1, degp):
    return pl.pallas_call(
        _tc1_body,
        grid=(NRT,),
        in_specs=[
            pl.BlockSpec((RT, D), lambda j: (j, 0)),
            pl.BlockSpec((D, D), lambda j: (0, 0)),
            pl.BlockSpec((NW, RT), lambda j: (0, j)),
        ],
        out_specs=[
            pl.BlockSpec((RT, H), lambda j: (j, 0)),
            pl.BlockSpec((RT, H), lambda j: (j, 0)),
            pl.BlockSpec((RT, 1), lambda j: (j, 0)),
        ],
        out_shape=[
            jax.ShapeDtypeStruct((NP, H), jnp.float32),
            jax.ShapeDtypeStruct((NP, H), jnp.float32),
            jax.ShapeDtypeStruct((NP, 1), jnp.float32),
        ],
    )(x, W1, degp)


def _stats_body(a0_ref, a1_ref, dis_ref, out_ref, scr):
    j = pl.program_id(0)
    dis = dis_ref[...]
    z = jnp.concatenate([a0_ref[...] * dis, a1_ref[...] * dis], axis=1)
    sums = jnp.stack([jnp.sum(z, axis=0), jnp.sum(z * z, axis=0)])

    @pl.when(j == 0)
    def _():
        scr[...] = jnp.zeros_like(scr)
    scr[...] += sums

    @pl.when(j == NRT - 1)
    def _():
        out_ref[...] = scr[...]


def _stats(a0, a1, dis2d):
    return pl.pallas_call(
        _stats_body,
        grid=(NRT,),
        in_specs=[
            pl.BlockSpec((RT, H), lambda j: (j, 0)),
            pl.BlockSpec((RT, H), lambda j: (j, 0)),
            pl.BlockSpec((RT, 1), lambda j: (j, 0)),
        ],
        out_specs=pl.BlockSpec((2, D), lambda j: (0, 0)),
        out_shape=jax.ShapeDtypeStruct((2, D), jnp.float32),
        scratch_shapes=[pltpu.VMEM((2, D), jnp.float32)],
    )(a0, a1, dis2d)


def _bn_relu(a0_ref, a1_ref, dis_ref, st_ref, g_ref, be_ref):
    dis = dis_ref[...]
    z = jnp.concatenate([a0_ref[...] * dis, a1_ref[...] * dis], axis=1)
    m = st_ref[0:1, :] * (1.0 / N)
    v = st_ref[1:2, :] * (1.0 / N) - m * m
    y = (z - m) * lax.rsqrt(v + 1e-5) * g_ref[...] + be_ref[...]
    return jnp.maximum(y, 0.0)


def _tcmid_body(a0_ref, a1_ref, dis_ref, st_ref, g_ref, be_ref, w_ref,
                h0_ref, h1_ref):
    j = pl.program_id(0)
    y = _bn_relu(a0_ref, a1_ref, dis_ref, st_ref, g_ref, be_ref)
    mm = jnp.dot(y, w_ref[...], preferred_element_type=jnp.float32)
    dis = dis_ref[...]
    row = j * RT + lax.broadcasted_iota(jnp.int32, (RT, 1), 0)
    hp = jnp.where(row < N, mm * dis, 0.0)
    h0_ref[...] = hp[:, :H]
    h1_ref[...] = hp[:, H:]


def _tcmid(a0, a1, dis2d, st, g, be, W2):
    return pl.pallas_call(
        _tcmid_body,
        grid=(NRT,),
        in_specs=[
            pl.BlockSpec((RT, H), lambda j: (j, 0)),
            pl.BlockSpec((RT, H), lambda j: (j, 0)),
            pl.BlockSpec((RT, 1), lambda j: (j, 0)),
            pl.BlockSpec((2, D), lambda j: (0, 0)),
            pl.BlockSpec((1, D), lambda j: (0, 0)),
            pl.BlockSpec((1, D), lambda j: (0, 0)),
            pl.BlockSpec((D, D), lambda j: (0, 0)),
        ],
        out_specs=[
            pl.BlockSpec((RT, H), lambda j: (j, 0)),
            pl.BlockSpec((RT, H), lambda j: (j, 0)),
        ],
        out_shape=[
            jax.ShapeDtypeStruct((NP, H), jnp.float32),
            jax.ShapeDtypeStruct((NP, H), jnp.float32),
        ],
    )(a0, a1, dis2d, st, g, be, W2)


def _tcfin_body(a0_ref, a1_ref, dis_ref, st_ref, g_ref, be_ref, out_ref):
    out_ref[...] = _bn_relu(a0_ref, a1_ref, dis_ref, st_ref, g_ref, be_ref)


def _tcfin(a0, a1, dis2d, st, g, be):
    return pl.pallas_call(
        _tcfin_body,
        grid=(NRT,),
        in_specs=[
            pl.BlockSpec((RT, H), lambda j: (j, 0)),
            pl.BlockSpec((RT, H), lambda j: (j, 0)),
            pl.BlockSpec((RT, 1), lambda j: (j, 0)),
            pl.BlockSpec((2, D), lambda j: (0, 0)),
            pl.BlockSpec((1, D), lambda j: (0, 0)),
            pl.BlockSpec((1, D), lambda j: (0, 0)),
        ],
        out_specs=pl.BlockSpec((RT, D), lambda j: (j, 0)),
        out_shape=jax.ShapeDtypeStruct((N, D), jnp.float32),
    )(a0, a1, dis2d, st, g, be)


def kernel(x, support, W1, b1, g1, be1, W2, b2, g2, be2):
    del b1, b2
    src = support[0].astype(jnp.int32)
    dst = support[1].astype(jnp.int32)
    g1r = g1.reshape(1, D)
    be1r = be1.reshape(1, D)
    g2r = g2.reshape(1, D)
    be2r = be2.reshape(1, D)

    srcp = jnp.concatenate([src, jnp.zeros((EP - E,), jnp.int32)]).reshape(NCH, CH)
    pad_dst = NP + (jnp.arange(EP - E, dtype=jnp.int32) % CH)
    dstp = jnp.concatenate([dst, pad_dst]).reshape(NCH, CH)

    degp = _deg_kernel(dst)
    h0, h1, dis2d = _tc1(x, W1, degp)
    a0, a1 = _agg_kernel(h0, h1, srcp, dstp)
    st1 = _stats(a0, a1, dis2d)
    h20, h21 = _tcmid(a0, a1, dis2d, st1, g1r, be1r, W2)
    b0u, b1u = _agg_kernel(h20, h21, srcp, dstp)
    st2 = _stats(b0u, b1u, dis2d)
    return _tcfin(b0u, b1u, dis2d, st2, g2r, be2r)


# speedup vs baseline: 1.5328x; 1.5328x over previous
import functools
import jax
import jax.numpy as jnp
from jax import lax
from jax.experimental import pallas as pl
from jax.experimental.pallas import tpu as pltpu
from jax.experimental.pallas import tpu_sc as plsc

N = 10000
E = 160000
D = 256
H = 128
NP = 10240
RT = 512
NRT = NP // RT
K = 128
NCHUNK = E // K
NTILES = 16
NW = 32
EPW = E // NW

_mesh = lambda: plsc.VectorSubcoreMesh(core_axis_name="c", subcore_axis_name="s")


def _deg_body(dst_hbm, out_hbm, dbuf, acc):
    c = lax.axis_index("c")
    s = lax.axis_index("s")
    w = s * 2 + c

    def zero(i, _):
        acc[pl.ds(i * 16, 16)] = jnp.zeros((16,), jnp.float32)
        return 0
    lax.fori_loop(0, NP // 16, zero, 0)

    dbuf[pl.ds(EPW - 16, 16)] = jnp.zeros((16,), jnp.int32)
    dbuf[pl.ds(EPW, 16)] = jnp.zeros((16,), jnp.int32)
    pltpu.sync_copy(dst_hbm.at[pl.ds(w * EPW, EPW)], dbuf.at[pl.ds(0, EPW)])

    ones = jnp.full((16,), 1.0, jnp.float32)

    def body(i, _):
        idx = dbuf[pl.ds(i * 16, 16)]
        plsc.addupdate_scatter(acc, [idx], ones)
        return 0
    lax.fori_loop(0, EPW // 16, body, 0)
    idx = dbuf[pl.ds((EPW // 16) * 16, 16)]
    mask = lax.broadcasted_iota(jnp.int32, (16,), 0) < (EPW % 16)
    plsc.addupdate_scatter(acc, [idx], ones, mask=mask)

    pltpu.sync_copy(acc, out_hbm.at[w])


@functools.partial(
    pl.kernel,
    out_type=jax.ShapeDtypeStruct((NW, NP), jnp.float32),
    mesh=_mesh(),
    scratch_types=[
        pltpu.VMEM((EPW + 16,), jnp.int32),
        pltpu.VMEM((NP,), jnp.float32),
    ],
    compiler_params=pltpu.CompilerParams(needs_layout_passes=False),
)
def _deg_kernel(dst_hbm, out_hbm, dbuf, acc):
    _deg_body(dst_hbm, out_hbm, dbuf, acc)


K = 128
NCHUNK = E // K


def _agg_init(tbl, acc, s, rows_v):
    for k in range(NP // NTILES // K):
        r = s * (NP // NTILES) + k * K
        pltpu.sync_copy(tbl.at[pl.ds(r, K)], rows_v)
        pltpu.sync_copy(rows_v, acc.at[pl.ds(r, K)])


def _agg_edges(tbl, acc, s, src_hbm, dst_hbm, src_i, dst_i, rows_v):
    n = 78 + jnp.where(s < NCHUNK - 78 * NTILES, 1, 0)

    def body(i, _):
        base = (s + NTILES * i) * K
        pltpu.sync_copy(src_hbm.at[pl.ds(base, K)], src_i.at[0])
        pltpu.sync_copy(dst_hbm.at[pl.ds(base, K)], dst_i.at[0])
        pltpu.sync_copy(tbl.at[src_i.at[0]], rows_v)
        pltpu.sync_copy(rows_v, acc.at[dst_i.at[0]], add=True)
        return 0
    lax.fori_loop(0, n, body, 0)


def _agg_writeout(out, acc, s, rows_v):
    for k in range(NP // NTILES // K):
        r = s * (NP // NTILES) + k * K
        pltpu.sync_copy(acc.at[pl.ds(r, K)], rows_v)
        pltpu.sync_copy(rows_v, out.at[pl.ds(r, K)])


@functools.partial(
    pl.kernel,
    out_type=(
        jax.ShapeDtypeStruct((NP, H), jnp.float32),
        jax.ShapeDtypeStruct((NP, H), jnp.float32),
    ),
    mesh=_mesh(),
    scratch_types=[
        pltpu.VMEM((1, K), jnp.int32),
        pltpu.VMEM((1, K), jnp.int32),
        pltpu.VMEM((K, H), jnp.float32),
        pltpu.VMEM_SHARED((NP, H), jnp.float32),
    ],
    compiler_params=pltpu.CompilerParams(needs_layout_passes=False),
)
def _agg_kernel(h0, h1, src_hbm, dst_hbm, o0, o1, src_i, dst_i, rows_v, acc):
    c = lax.axis_index("c")
    s = lax.axis_index("s")

    @pl.when(c == 0)
    def _():
        _agg_init(h0, acc, s, rows_v)

    @pl.when(c == 1)
    def _():
        _agg_init(h1, acc, s, rows_v)

    plsc.subcore_barrier()

    @pl.when(c == 0)
    def _():
        _agg_edges(h0, acc, s, src_hbm, dst_hbm, src_i, dst_i, rows_v)

    @pl.when(c == 1)
    def _():
        _agg_edges(h1, acc, s, src_hbm, dst_hbm, src_i, dst_i, rows_v)

    plsc.subcore_barrier()

    @pl.when(c == 0)
    def _():
        _agg_writeout(o0, acc, s, rows_v)

    @pl.when(c == 1)
    def _():
        _agg_writeout(o1, acc, s, rows_v)


def _tc1_body(x_ref, w_ref, degp_ref, h0_ref, h1_ref, dis_ref):
    j = pl.program_id(0)
    deg = jnp.sum(degp_ref[...], axis=0) + 1.0
    dis = lax.rsqrt(deg)
    mm = jnp.dot(x_ref[...], w_ref[...], preferred_element_type=jnp.float32)
    row = j * RT + lax.broadcasted_iota(jnp.int32, (RT, 1), 0)
    hp = jnp.where(row < N, mm * dis[:, None], 0.0)
    h0_ref[...] = hp[:, :H]
    h1_ref[...] = hp[:, H:]
    dis_ref[...] = dis[:, None]


def _tc1(x, W1, degp):
    return pl.pallas_call(
        _tc1_body,
        grid=(NRT,),
        in_specs=[
            pl.BlockSpec((RT, D), lambda j: (j, 0)),
            pl.BlockSpec((D, D), lambda j: (0, 0)),
            pl.BlockSpec((NW, RT), lambda j: (0, j)),
        ],
        out_specs=[
            pl.BlockSpec((RT, H), lambda j: (j, 0)),
            pl.BlockSpec((RT, H), lambda j: (j, 0)),
            pl.BlockSpec((RT, 1), lambda j: (j, 0)),
        ],
        out_shape=[
            jax.ShapeDtypeStruct((NP, H), jnp.float32),
            jax.ShapeDtypeStruct((NP, H), jnp.float32),
            jax.ShapeDtypeStruct((NP, 1), jnp.float32),
        ],
    )(x, W1, degp)


def _bn_accum(j, z, scr):
    @pl.when(j == 0)
    def _():
        scr[...] = jnp.zeros_like(scr)

    @pl.when(j < NRT)
    def _():
        scr[...] += jnp.stack([jnp.sum(z, axis=0), jnp.sum(z * z, axis=0)])


def _bn_apply(z, scr, g_ref, be_ref):
    m = scr[0:1, :] * (1.0 / N)
    v = scr[1:2, :] * (1.0 / N) - m * m
    return jnp.maximum((z - m) * lax.rsqrt(v + 1e-5) * g_ref[...]
                       + be_ref[...], 0.0)


def _tcmid_body(a0_ref, a1_ref, dis_ref, g_ref, be_ref, w_ref,
                h0_ref, h1_ref, scr):
    j = pl.program_id(0)
    dis = dis_ref[...]
    z = jnp.concatenate([a0_ref[...] * dis, a1_ref[...] * dis], axis=1)
    _bn_accum(j, z, scr)

    @pl.when(j >= NRT)
    def _():
        y = _bn_apply(z, scr, g_ref, be_ref)
        mm = jnp.dot(y, w_ref[...], preferred_element_type=jnp.float32)
        row = (j - NRT) * RT + lax.broadcasted_iota(jnp.int32, (RT, 1), 0)
        hp = jnp.where(row < N, mm * dis, 0.0)
        h0_ref[...] = hp[:, :H]
        h1_ref[...] = hp[:, H:]


def _tcmid(a0, a1, dis2d, g, be, W2):
    blk = lambda j: (lax.rem(j, NRT), 0)
    out_blk = lambda j: (jnp.maximum(j - NRT, 0), 0)
    return pl.pallas_call(
        _tcmid_body,
        grid=(2 * NRT,),
        in_specs=[
            pl.BlockSpec((RT, H), blk),
            pl.BlockSpec((RT, H), blk),
            pl.BlockSpec((RT, 1), blk),
            pl.BlockSpec((1, D), lambda j: (0, 0)),
            pl.BlockSpec((1, D), lambda j: (0, 0)),
            pl.BlockSpec((D, D), lambda j: (0, 0)),
        ],
        out_specs=[
            pl.BlockSpec((RT, H), out_blk),
            pl.BlockSpec((RT, H), out_blk),
        ],
        out_shape=[
            jax.ShapeDtypeStruct((NP, H), jnp.float32),
            jax.ShapeDtypeStruct((NP, H), jnp.float32),
        ],
        scratch_shapes=[pltpu.VMEM((2, D), jnp.float32)],
    )(a0, a1, dis2d, g, be, W2)


def _tcfin_body(a0_ref, a1_ref, dis_ref, g_ref, be_ref, out_ref, scr):
    j = pl.program_id(0)
    dis = dis_ref[...]
    z = jnp.concatenate([a0_ref[...] * dis, a1_ref[...] * dis], axis=1)
    _bn_accum(j, z, scr)

    @pl.when(j >= NRT)
    def _():
        out_ref[...] = _bn_apply(z, scr, g_ref, be_ref)


def _tcfin(a0, a1, dis2d, g, be):
    blk = lambda j: (lax.rem(j, NRT), 0)
    out_blk = lambda j: (jnp.maximum(j - NRT, 0), 0)
    return pl.pallas_call(
        _tcfin_body,
        grid=(2 * NRT,),
        in_specs=[
            pl.BlockSpec((RT, H), blk),
            pl.BlockSpec((RT, H), blk),
            pl.BlockSpec((RT, 1), blk),
            pl.BlockSpec((1, D), lambda j: (0, 0)),
            pl.BlockSpec((1, D), lambda j: (0, 0)),
        ],
        out_specs=pl.BlockSpec((RT, D), out_blk),
        out_shape=jax.ShapeDtypeStruct((N, D), jnp.float32),
        scratch_shapes=[pltpu.VMEM((2, D), jnp.float32)],
    )(a0, a1, dis2d, g, be)


def kernel(x, support, W1, b1, g1, be1, W2, b2, g2, be2):
    del b1, b2
    src = support[0].astype(jnp.int32)
    dst = support[1].astype(jnp.int32)
    g1r = g1.reshape(1, D)
    be1r = be1.reshape(1, D)
    g2r = g2.reshape(1, D)
    be2r = be2.reshape(1, D)

    degp = _deg_kernel(dst)
    h0, h1, dis2d = _tc1(x, W1, degp)
    a0, a1 = _agg_kernel(h0, h1, src, dst)
    h20, h21 = _tcmid(a0, a1, dis2d, g1r, be1r, W2)
    b0u, b1u = _agg_kernel(h20, h21, src, dst)
    return _tcfin(b0u, b1u, dis2d, g2r, be2r)


# speedup vs baseline: 1.7168x; 1.1200x over previous
import functools
import jax
import jax.numpy as jnp
from jax import lax
from jax.experimental import pallas as pl
from jax.experimental.pallas import tpu as pltpu
from jax.experimental.pallas import tpu_sc as plsc

N = 10000
E = 160000
D = 256
H = 128
NP = 10240
RT = 512
NRT = NP // RT
K = 128
NCHUNK = E // K
NTILES = 16
NW = 32
EPW = E // NW

_mesh = lambda: plsc.VectorSubcoreMesh(core_axis_name="c", subcore_axis_name="s")


def _deg_body(dst_hbm, out_hbm, dbuf, acc):
    c = lax.axis_index("c")
    s = lax.axis_index("s")
    w = s * 2 + c

    def zero(i, _):
        acc[pl.ds(i * 16, 16)] = jnp.zeros((16,), jnp.float32)
        return 0
    lax.fori_loop(0, NP // 16, zero, 0)

    dbuf[pl.ds(EPW - 16, 16)] = jnp.zeros((16,), jnp.int32)
    dbuf[pl.ds(EPW, 16)] = jnp.zeros((16,), jnp.int32)
    pltpu.sync_copy(dst_hbm.at[pl.ds(w * EPW, EPW)], dbuf.at[pl.ds(0, EPW)])

    ones = jnp.full((16,), 1.0, jnp.float32)

    def body(i, _):
        idx = dbuf[pl.ds(i * 16, 16)]
        plsc.addupdate_scatter(acc, [idx], ones)
        return 0
    lax.fori_loop(0, EPW // 16, body, 0)
    idx = dbuf[pl.ds((EPW // 16) * 16, 16)]
    mask = lax.broadcasted_iota(jnp.int32, (16,), 0) < (EPW % 16)
    plsc.addupdate_scatter(acc, [idx], ones, mask=mask)

    pltpu.sync_copy(acc, out_hbm.at[w])


@functools.partial(
    pl.kernel,
    out_type=jax.ShapeDtypeStruct((NW, NP), jnp.float32),
    mesh=_mesh(),
    scratch_types=[
        pltpu.VMEM((EPW + 16,), jnp.int32),
        pltpu.VMEM((NP,), jnp.float32),
    ],
    compiler_params=pltpu.CompilerParams(needs_layout_passes=False),
)
def _deg_kernel(dst_hbm, out_hbm, dbuf, acc):
    _deg_body(dst_hbm, out_hbm, dbuf, acc)


K = 128
NCHUNK = E // K


def _agg_init(tbl, acc, s, rows_v):
    for k in range(NP // NTILES // K):
        r = s * (NP // NTILES) + k * K
        pltpu.sync_copy(tbl.at[pl.ds(r, K)], rows_v)
        pltpu.sync_copy(rows_v, acc.at[pl.ds(r, K)])


def _agg_edges(tbl, acc, s, sd_hbm, sd_i, rows_v):
    n = 78 + jnp.where(s < NCHUNK - 78 * NTILES, 1, 0)

    def body(i, _):
        chunk = s + NTILES * i
        pltpu.sync_copy(sd_hbm.at[chunk], sd_i)
        pltpu.sync_copy(tbl.at[sd_i.at[0]], rows_v)
        pltpu.sync_copy(rows_v, acc.at[sd_i.at[1]], add=True)
        return 0
    lax.fori_loop(0, n, body, 0)


def _agg_writeout(out, acc, s, rows_v):
    for k in range(NP // NTILES // K):
        r = s * (NP // NTILES) + k * K
        pltpu.sync_copy(acc.at[pl.ds(r, K)], rows_v)
        pltpu.sync_copy(rows_v, out.at[pl.ds(r, K)])


@functools.partial(
    pl.kernel,
    out_type=(
        jax.ShapeDtypeStruct((NP, H), jnp.float32),
        jax.ShapeDtypeStruct((NP, H), jnp.float32),
    ),
    mesh=_mesh(),
    scratch_types=[
        pltpu.VMEM((2, K), jnp.int32),
        pltpu.VMEM((K, H), jnp.float32),
        pltpu.VMEM_SHARED((NP, H), jnp.float32),
    ],
    compiler_params=pltpu.CompilerParams(needs_layout_passes=False),
)
def _agg_kernel(h0, h1, sd_hbm, o0, o1, sd_i, rows_v, acc):
    c = lax.axis_index("c")
    s = lax.axis_index("s")

    @pl.when(c == 0)
    def _():
        _agg_init(h0, acc, s, rows_v)

    @pl.when(c == 1)
    def _():
        _agg_init(h1, acc, s, rows_v)

    plsc.subcore_barrier()

    @pl.when(c == 0)
    def _():
        _agg_edges(h0, acc, s, sd_hbm, sd_i, rows_v)

    @pl.when(c == 1)
    def _():
        _agg_edges(h1, acc, s, sd_hbm, sd_i, rows_v)

    plsc.subcore_barrier()

    @pl.when(c == 0)
    def _():
        _agg_writeout(o0, acc, s, rows_v)

    @pl.when(c == 1)
    def _():
        _agg_writeout(o1, acc, s, rows_v)


def _tc1_body(x_ref, w_ref, degp_ref, h0_ref, h1_ref, dis_ref):
    j = pl.program_id(0)
    deg = jnp.sum(degp_ref[...], axis=0) + 1.0
    dis = lax.rsqrt(deg)
    mm = jnp.dot(x_ref[...], w_ref[...], preferred_element_type=jnp.float32)
    row = j * RT + lax.broadcasted_iota(jnp.int32, (RT, 1), 0)
    hp = jnp.where(row < N, mm * dis[:, None], 0.0)
    h0_ref[...] = hp[:, :H]
    h1_ref[...] = hp[:, H:]
    dis_ref[...] = dis[:, None]


def _tc1(x, W1, degp):
    return pl.pallas_call(
        _tc1_body,
        grid=(NRT,),
        in_specs=[
            pl.BlockSpec((RT, D), lambda j: (j, 0)),
            pl.BlockSpec((D, D), lambda j: (0, 0)),
            pl.BlockSpec((NW, RT), lambda j: (0, j)),
        ],
        out_specs=[
            pl.BlockSpec((RT, H), lambda j: (j, 0)),
            pl.BlockSpec((RT, H), lambda j: (j, 0)),
            pl.BlockSpec((RT, 1), lambda j: (j, 0)),
        ],
        out_shape=[
            jax.ShapeDtypeStruct((NP, H), jnp.float32),
            jax.ShapeDtypeStruct((NP, H), jnp.float32),
            jax.ShapeDtypeStruct((NP, 1), jnp.float32),
        ],
    )(x, W1, degp)


def _bn_accum(j, z, scr):
    @pl.when(j == 0)
    def _():
        scr[...] = jnp.zeros_like(scr)

    @pl.when(j < NRT)
    def _():
        scr[...] += jnp.stack([jnp.sum(z, axis=0), jnp.sum(z * z, axis=0)])


def _bn_apply(z, scr, g_ref, be_ref):
    m = scr[0:1, :] * (1.0 / N)
    v = scr[1:2, :] * (1.0 / N) - m * m
    return jnp.maximum((z - m) * lax.rsqrt(v + 1e-5) * g_ref[...]
                       + be_ref[...], 0.0)


def _tcmid_body(a0_ref, a1_ref, dis_ref, g_ref, be_ref, w_ref,
                h0_ref, h1_ref, scr):
    j = pl.program_id(0)
    dis = dis_ref[...]
    z = jnp.concatenate([a0_ref[...] * dis, a1_ref[...] * dis], axis=1)
    _bn_accum(j, z, scr)

    @pl.when(j >= NRT)
    def _():
        y = _bn_apply(z, scr, g_ref, be_ref)
        mm = jnp.dot(y, w_ref[...], preferred_element_type=jnp.float32)
        row = (j - NRT) * RT + lax.broadcasted_iota(jnp.int32, (RT, 1), 0)
        hp = jnp.where(row < N, mm * dis, 0.0)
        h0_ref[...] = hp[:, :H]
        h1_ref[...] = hp[:, H:]


def _tcmid(a0, a1, dis2d, g, be, W2):
    blk = lambda j: (lax.rem(j, NRT), 0)
    out_blk = lambda j: (jnp.maximum(j - NRT, 0), 0)
    return pl.pallas_call(
        _tcmid_body,
        grid=(2 * NRT,),
        in_specs=[
            pl.BlockSpec((RT, H), blk),
            pl.BlockSpec((RT, H), blk),
            pl.BlockSpec((RT, 1), blk),
            pl.BlockSpec((1, D), lambda j: (0, 0)),
            pl.BlockSpec((1, D), lambda j: (0, 0)),
            pl.BlockSpec((D, D), lambda j: (0, 0)),
        ],
        out_specs=[
            pl.BlockSpec((RT, H), out_blk),
            pl.BlockSpec((RT, H), out_blk),
        ],
        out_shape=[
            jax.ShapeDtypeStruct((NP, H), jnp.float32),
            jax.ShapeDtypeStruct((NP, H), jnp.float32),
        ],
        scratch_shapes=[pltpu.VMEM((2, D), jnp.float32)],
    )(a0, a1, dis2d, g, be, W2)


def _tcfin_body(a0_ref, a1_ref, dis_ref, g_ref, be_ref, out_ref, scr):
    j = pl.program_id(0)
    dis = dis_ref[...]
    z = jnp.concatenate([a0_ref[...] * dis, a1_ref[...] * dis], axis=1)
    _bn_accum(j, z, scr)

    @pl.when(j >= NRT)
    def _():
        out_ref[...] = _bn_apply(z, scr, g_ref, be_ref)


def _tcfin(a0, a1, dis2d, g, be):
    blk = lambda j: (lax.rem(j, NRT), 0)
    out_blk = lambda j: (jnp.maximum(j - NRT, 0), 0)
    return pl.pallas_call(
        _tcfin_body,
        grid=(2 * NRT,),
        in_specs=[
            pl.BlockSpec((RT, H), blk),
            pl.BlockSpec((RT, H), blk),
            pl.BlockSpec((RT, 1), blk),
            pl.BlockSpec((1, D), lambda j: (0, 0)),
            pl.BlockSpec((1, D), lambda j: (0, 0)),
        ],
        out_specs=pl.BlockSpec((RT, D), out_blk),
        out_shape=jax.ShapeDtypeStruct((N, D), jnp.float32),
        scratch_shapes=[pltpu.VMEM((2, D), jnp.float32)],
    )(a0, a1, dis2d, g, be)


def kernel(x, support, W1, b1, g1, be1, W2, b2, g2, be2):
    del b1, b2
    src = support[0].astype(jnp.int32)
    dst = support[1].astype(jnp.int32)
    g1r = g1.reshape(1, D)
    be1r = be1.reshape(1, D)
    g2r = g2.reshape(1, D)
    be2r = be2.reshape(1, D)

    sd = jnp.stack([src.reshape(NCHUNK, K), dst.reshape(NCHUNK, K)], axis=1)

    degp = _deg_kernel(dst)
    h0, h1, dis2d = _tc1(x, W1, degp)
    a0, a1 = _agg_kernel(h0, h1, sd)
    h20, h21 = _tcmid(a0, a1, dis2d, g1r, be1r, W2)
    b0u, b1u = _agg_kernel(h20, h21, sd)
    return _tcfin(b0u, b1u, dis2d, g2r, be2r)


# speedup vs baseline: 1.9854x; 1.1565x over previous
import functools
import jax
import jax.numpy as jnp
from jax import lax
from jax.experimental import pallas as pl
from jax.experimental.pallas import tpu as pltpu
from jax.experimental.pallas import tpu_sc as plsc

N = 10000
E = 160000
D = 256
H = 128
NP = 10240
RT = 512
NRT = NP // RT
K = 128
NCHUNK = E // K
NTILES = 16
NW = 32
EPW = E // NW

_mesh = lambda: plsc.VectorSubcoreMesh(core_axis_name="c", subcore_axis_name="s")


def _deg_body(dst_hbm, out_hbm, dbuf, acc):
    c = lax.axis_index("c")
    s = lax.axis_index("s")
    w = s * 2 + c

    def zero(i, _):
        acc[pl.ds(i * 16, 16)] = jnp.zeros((16,), jnp.float32)
        return 0
    lax.fori_loop(0, NP // 16, zero, 0)

    dbuf[pl.ds(EPW - 16, 16)] = jnp.zeros((16,), jnp.int32)
    dbuf[pl.ds(EPW, 16)] = jnp.zeros((16,), jnp.int32)
    pltpu.sync_copy(dst_hbm.at[pl.ds(w * EPW, EPW)], dbuf.at[pl.ds(0, EPW)])

    ones = jnp.full((16,), 1.0, jnp.float32)

    def body(i, _):
        idx = dbuf[pl.ds(i * 16, 16)]
        plsc.addupdate_scatter(acc, [idx], ones)
        return 0
    lax.fori_loop(0, EPW // 16, body, 0)
    idx = dbuf[pl.ds((EPW // 16) * 16, 16)]
    mask = lax.broadcasted_iota(jnp.int32, (16,), 0) < (EPW % 16)
    plsc.addupdate_scatter(acc, [idx], ones, mask=mask)

    pltpu.sync_copy(acc, out_hbm.at[w])


@functools.partial(
    pl.kernel,
    out_type=jax.ShapeDtypeStruct((NW, NP), jnp.float32),
    mesh=_mesh(),
    scratch_types=[
        pltpu.VMEM((EPW + 16,), jnp.int32),
        pltpu.VMEM((NP,), jnp.float32),
    ],
    compiler_params=pltpu.CompilerParams(needs_layout_passes=False),
)
def _deg_kernel(dst_hbm, out_hbm, dbuf, acc):
    _deg_body(dst_hbm, out_hbm, dbuf, acc)


K = 128
NCHUNK = E // K


def _agg_init(tbl, acc, s, rows_v):
    for k in range(NP // NTILES // K):
        r = s * (NP // NTILES) + k * K
        pltpu.sync_copy(tbl.at[pl.ds(r, K)], rows_v)
        pltpu.sync_copy(rows_v, acc.at[pl.ds(r, K)])


def _agg_edges(tbl, acc, s, sd_hbm, sdA, sdB, rows_v, ixa, ixb):
    n = 78 + jnp.where(s < NCHUNK - 78 * NTILES, 1, 0)

    def load(i, buf, sem):
        pltpu.async_copy(sd_hbm.at[s + NTILES * i], buf, sem)

    def wait(buf, sem):
        pltpu.make_async_copy(sd_hbm.at[0], buf, sem).wait()

    def work(buf):
        pltpu.sync_copy(tbl.at[buf.at[0]], rows_v)
        pltpu.sync_copy(rows_v, acc.at[buf.at[1]], add=True)

    load(0, sdA, ixa)
    load(1, sdB, ixb)

    def body(j, _):
        wait(sdA, ixa)
        work(sdA)

        @pl.when(2 * j + 2 < n)
        def _():
            load(2 * j + 2, sdA, ixa)

        wait(sdB, ixb)
        work(sdB)

        @pl.when(2 * j + 3 < n)
        def _():
            load(2 * j + 3, sdB, ixb)
        return 0
    lax.fori_loop(0, n // 2, body, 0)

    @pl.when(n % 2 == 1)
    def _():
        wait(sdA, ixa)
        work(sdA)


def _agg_writeout(out, acc, s, rows_v):
    for k in range(NP // NTILES // K):
        r = s * (NP // NTILES) + k * K
        pltpu.sync_copy(acc.at[pl.ds(r, K)], rows_v)
        pltpu.sync_copy(rows_v, out.at[pl.ds(r, K)])


@functools.partial(
    pl.kernel,
    out_type=(
        jax.ShapeDtypeStruct((NP, H), jnp.float32),
        jax.ShapeDtypeStruct((NP, H), jnp.float32),
    ),
    mesh=_mesh(),
    scratch_types=[
        pltpu.VMEM((2, K), jnp.int32),
        pltpu.VMEM((2, K), jnp.int32),
        pltpu.VMEM((K, H), jnp.float32),
        pltpu.VMEM_SHARED((NP, H), jnp.float32),
        pltpu.SemaphoreType.DMA,
        pltpu.SemaphoreType.DMA,
    ],
    compiler_params=pltpu.CompilerParams(needs_layout_passes=False),
)
def _agg_kernel(h0, h1, sd_hbm, o0, o1, sdA, sdB, rows_v, acc, ixa, ixb):
    c = lax.axis_index("c")
    s = lax.axis_index("s")

    @pl.when(c == 0)
    def _():
        _agg_init(h0, acc, s, rows_v)

    @pl.when(c == 1)
    def _():
        _agg_init(h1, acc, s, rows_v)

    plsc.subcore_barrier()

    @pl.when(c == 0)
    def _():
        _agg_edges(h0, acc, s, sd_hbm, sdA, sdB, rows_v, ixa, ixb)

    @pl.when(c == 1)
    def _():
        _agg_edges(h1, acc, s, sd_hbm, sdA, sdB, rows_v, ixa, ixb)

    plsc.subcore_barrier()

    @pl.when(c == 0)
    def _():
        _agg_writeout(o0, acc, s, rows_v)

    @pl.when(c == 1)
    def _():
        _agg_writeout(o1, acc, s, rows_v)


def _tc1_body(x_ref, w_ref, degp_ref, h0_ref, h1_ref, dis_ref):
    j = pl.program_id(0)
    deg = jnp.sum(degp_ref[...], axis=0) + 1.0
    dis = lax.rsqrt(deg)
    mm = jnp.dot(x_ref[...], w_ref[...], preferred_element_type=jnp.float32)
    row = j * RT + lax.broadcasted_iota(jnp.int32, (RT, 1), 0)
    hp = jnp.where(row < N, mm * dis[:, None], 0.0)
    h0_ref[...] = hp[:, :H]
    h1_ref[...] = hp[:, H:]
    dis_ref[...] = dis[:, None]


def _tc1(x, W1, degp):
    return pl.pallas_call(
        _tc1_body,
        grid=(NRT,),
        in_specs=[
            pl.BlockSpec((RT, D), lambda j: (j, 0)),
            pl.BlockSpec((D, D), lambda j: (0, 0)),
            pl.BlockSpec((NW, RT), lambda j: (0, j)),
        ],
        out_specs=[
            pl.BlockSpec((RT, H), lambda j: (j, 0)),
            pl.BlockSpec((RT, H), lambda j: (j, 0)),
            pl.BlockSpec((RT, 1), lambda j: (j, 0)),
        ],
        out_shape=[
            jax.ShapeDtypeStruct((NP, H), jnp.float32),
            jax.ShapeDtypeStruct((NP, H), jnp.float32),
            jax.ShapeDtypeStruct((NP, 1), jnp.float32),
        ],
    )(x, W1, degp)


def _bn_accum(j, z, scr):
    @pl.when(j == 0)
    def _():
        scr[...] = jnp.zeros_like(scr)

    @pl.when(j < NRT)
    def _():
        scr[...] += jnp.stack([jnp.sum(z, axis=0), jnp.sum(z * z, axis=0)])


def _bn_apply(z, scr, g_ref, be_ref):
    m = scr[0:1, :] * (1.0 / N)
    v = scr[1:2, :] * (1.0 / N) - m * m
    return jnp.maximum((z - m) * lax.rsqrt(v + 1e-5) * g_ref[...]
                       + be_ref[...], 0.0)


def _tcmid_body(a0_ref, a1_ref, dis_ref, g_ref, be_ref, w_ref,
                h0_ref, h1_ref, scr):
    j = pl.program_id(0)
    dis = dis_ref[...]
    z = jnp.concatenate([a0_ref[...] * dis, a1_ref[...] * dis], axis=1)
    _bn_accum(j, z, scr)

    @pl.when(j >= NRT)
    def _():
        y = _bn_apply(z, scr, g_ref, be_ref)
        mm = jnp.dot(y, w_ref[...], preferred_element_type=jnp.float32)
        row = (j - NRT) * RT + lax.broadcasted_iota(jnp.int32, (RT, 1), 0)
        hp = jnp.where(row < N, mm * dis, 0.0)
        h0_ref[...] = hp[:, :H]
        h1_ref[...] = hp[:, H:]


def _tcmid(a0, a1, dis2d, g, be, W2):
    blk = lambda j: (lax.rem(j, NRT), 0)
    out_blk = lambda j: (jnp.maximum(j - NRT, 0), 0)
    return pl.pallas_call(
        _tcmid_body,
        grid=(2 * NRT,),
        in_specs=[
            pl.BlockSpec((RT, H), blk),
            pl.BlockSpec((RT, H), blk),
            pl.BlockSpec((RT, 1), blk),
            pl.BlockSpec((1, D), lambda j: (0, 0)),
            pl.BlockSpec((1, D), lambda j: (0, 0)),
            pl.BlockSpec((D, D), lambda j: (0, 0)),
        ],
        out_specs=[
            pl.BlockSpec((RT, H), out_blk),
            pl.BlockSpec((RT, H), out_blk),
        ],
        out_shape=[
            jax.ShapeDtypeStruct((NP, H), jnp.float32),
            jax.ShapeDtypeStruct((NP, H), jnp.float32),
        ],
        scratch_shapes=[pltpu.VMEM((2, D), jnp.float32)],
    )(a0, a1, dis2d, g, be, W2)


def _tcfin_body(a0_ref, a1_ref, dis_ref, g_ref, be_ref, out_ref, scr):
    j = pl.program_id(0)
    dis = dis_ref[...]
    z = jnp.concatenate([a0_ref[...] * dis, a1_ref[...] * dis], axis=1)
    _bn_accum(j, z, scr)

    @pl.when(j >= NRT)
    def _():
        out_ref[...] = _bn_apply(z, scr, g_ref, be_ref)


def _tcfin(a0, a1, dis2d, g, be):
    blk = lambda j: (lax.rem(j, NRT), 0)
    out_blk = lambda j: (jnp.maximum(j - NRT, 0), 0)
    return pl.pallas_call(
        _tcfin_body,
        grid=(2 * NRT,),
        in_specs=[
            pl.BlockSpec((RT, H), blk),
            pl.BlockSpec((RT, H), blk),
            pl.BlockSpec((RT, 1), blk),
            pl.BlockSpec((1, D), lambda j: (0, 0)),
            pl.BlockSpec((1, D), lambda j: (0, 0)),
        ],
        out_specs=pl.BlockSpec((RT, D), out_blk),
        out_shape=jax.ShapeDtypeStruct((N, D), jnp.float32),
        scratch_shapes=[pltpu.VMEM((2, D), jnp.float32)],
    )(a0, a1, dis2d, g, be)


def kernel(x, support, W1, b1, g1, be1, W2, b2, g2, be2):
    del b1, b2
    src = support[0].astype(jnp.int32)
    dst = support[1].astype(jnp.int32)
    g1r = g1.reshape(1, D)
    be1r = be1.reshape(1, D)
    g2r = g2.reshape(1, D)
    be2r = be2.reshape(1, D)

    sd = jnp.stack([src.reshape(NCHUNK, K), dst.reshape(NCHUNK, K)], axis=1)

    degp = _deg_kernel(dst)
    h0, h1, dis2d = _tc1(x, W1, degp)
    a0, a1 = _agg_kernel(h0, h1, sd)
    h20, h21 = _tcmid(a0, a1, dis2d, g1r, be1r, W2)
    b0u, b1u = _agg_kernel(h20, h21, sd)
    return _tcfin(b0u, b1u, dis2d, g2r, be2r)


# speedup vs baseline: 2.4637x; 1.2409x over previous
import functools
import jax
import jax.numpy as jnp
from jax import lax
from jax.experimental import pallas as pl
from jax.experimental.pallas import tpu as pltpu
from jax.experimental.pallas import tpu_sc as plsc

N = 10000
E = 160000
D = 256
H = 128
NP = 10240
RT = 512
NRT = NP // RT
K = 128
NCHUNK = E // K
NTILES = 16
NW = 32
EPW = E // NW

_mesh = lambda: plsc.VectorSubcoreMesh(core_axis_name="c", subcore_axis_name="s")


def _deg_body(dst_hbm, out_hbm, dbuf, acc):
    c = lax.axis_index("c")
    s = lax.axis_index("s")
    w = s * 2 + c

    def zero(i, _):
        acc[pl.ds(i * 16, 16)] = jnp.zeros((16,), jnp.float32)
        return 0
    lax.fori_loop(0, NP // 16, zero, 0)

    dbuf[pl.ds(EPW - 16, 16)] = jnp.zeros((16,), jnp.int32)
    dbuf[pl.ds(EPW, 16)] = jnp.zeros((16,), jnp.int32)
    pltpu.sync_copy(dst_hbm.at[pl.ds(w * EPW, EPW)], dbuf.at[pl.ds(0, EPW)])

    ones = jnp.full((16,), 1.0, jnp.float32)

    def body(i, _):
        idx = dbuf[pl.ds(i * 16, 16)]
        plsc.addupdate_scatter(acc, [idx], ones)
        return 0
    lax.fori_loop(0, EPW // 16, body, 0)
    idx = dbuf[pl.ds((EPW // 16) * 16, 16)]
    mask = lax.broadcasted_iota(jnp.int32, (16,), 0) < (EPW % 16)
    plsc.addupdate_scatter(acc, [idx], ones, mask=mask)

    pltpu.sync_copy(acc, out_hbm.at[w])


@functools.partial(
    pl.kernel,
    out_type=jax.ShapeDtypeStruct((NW, NP), jnp.float32),
    mesh=_mesh(),
    scratch_types=[
        pltpu.VMEM((EPW + 16,), jnp.int32),
        pltpu.VMEM((NP,), jnp.float32),
    ],
    compiler_params=pltpu.CompilerParams(needs_layout_passes=False),
)
def _deg_kernel(dst_hbm, out_hbm, dbuf, acc):
    _deg_body(dst_hbm, out_hbm, dbuf, acc)


K = 128
NCHUNK = E // K


def _agg_init(tbl, acc, s, rows_v):
    for k in range(NP // NTILES // K):
        r = s * (NP // NTILES) + k * K
        pltpu.sync_copy(tbl.at[pl.ds(r, K)], rows_v)
        pltpu.sync_copy(rows_v, acc.at[pl.ds(r, K)])


def _agg_edges(tbl, acc, s, sd_hbm, sdA, sdB, rowsA, rowsB,
               ixa, ixb, ga, gb):
    n = 78 + jnp.where(s < NCHUNK - 78 * NTILES, 1, 0)

    def load(i, buf, sem):
        pltpu.async_copy(sd_hbm.at[s + NTILES * i], buf, sem)

    def wait_ix(buf, sem):
        pltpu.make_async_copy(sd_hbm.at[0], buf, sem).wait()

    def gather(idx, rows, sem):
        pltpu.async_copy(tbl.at[idx.at[0]], rows, sem)

    def wait_g(rows, sem):
        pltpu.make_async_copy(tbl.at[sdA.at[0]], rows, sem).wait()

    def scatter(idx, rows):
        pltpu.sync_copy(rows, acc.at[idx.at[1]], add=True)

    load(0, sdA, ixa)
    load(1, sdB, ixb)
    wait_ix(sdA, ixa)
    gather(sdA, rowsA, ga)

    def body(j, _):
        wait_g(rowsA, ga)
        wait_ix(sdB, ixb)
        gather(sdB, rowsB, gb)
        scatter(sdA, rowsA)

        @pl.when(2 * j + 2 < n)
        def _():
            load(2 * j + 2, sdA, ixa)

        wait_g(rowsB, gb)

        @pl.when(2 * j + 2 < n)
        def _():
            wait_ix(sdA, ixa)
            gather(sdA, rowsA, ga)
        scatter(sdB, rowsB)

        @pl.when(2 * j + 3 < n)
        def _():
            load(2 * j + 3, sdB, ixb)
        return 0
    lax.fori_loop(0, n // 2, body, 0)

    @pl.when(n % 2 == 1)
    def _():
        wait_g(rowsA, ga)
        scatter(sdA, rowsA)


def _agg_writeout(out, acc, s, rows_v):
    for k in range(NP // NTILES // K):
        r = s * (NP // NTILES) + k * K
        pltpu.sync_copy(acc.at[pl.ds(r, K)], rows_v)
        pltpu.sync_copy(rows_v, out.at[pl.ds(r, K)])


@functools.partial(
    pl.kernel,
    out_type=(
        jax.ShapeDtypeStruct((NP, H), jnp.float32),
        jax.ShapeDtypeStruct((NP, H), jnp.float32),
    ),
    mesh=_mesh(),
    scratch_types=[
        pltpu.VMEM((2, K), jnp.int32),
        pltpu.VMEM((2, K), jnp.int32),
        pltpu.VMEM((K, H), jnp.float32),
        pltpu.VMEM((K, H), jnp.float32),
        pltpu.VMEM_SHARED((NP, H), jnp.float32),
        pltpu.SemaphoreType.DMA,
        pltpu.SemaphoreType.DMA,
        pltpu.SemaphoreType.DMA,
        pltpu.SemaphoreType.DMA,
    ],
    compiler_params=pltpu.CompilerParams(needs_layout_passes=False),
)
def _agg_kernel(h0, h1, sd_hbm, o0, o1, sdA, sdB, rowsA, rowsB, acc,
                ixa, ixb, ga, gb):
    c = lax.axis_index("c")
    s = lax.axis_index("s")

    @pl.when(c == 0)
    def _():
        _agg_init(h0, acc, s, rowsA)

    @pl.when(c == 1)
    def _():
        _agg_init(h1, acc, s, rowsA)

    plsc.subcore_barrier()

    @pl.when(c == 0)
    def _():
        _agg_edges(h0, acc, s, sd_hbm, sdA, sdB, rowsA, rowsB,
                   ixa, ixb, ga, gb)

    @pl.when(c == 1)
    def _():
        _agg_edges(h1, acc, s, sd_hbm, sdA, sdB, rowsA, rowsB,
                   ixa, ixb, ga, gb)

    plsc.subcore_barrier()

    @pl.when(c == 0)
    def _():
        _agg_writeout(o0, acc, s, rowsA)

    @pl.when(c == 1)
    def _():
        _agg_writeout(o1, acc, s, rowsA)


def _tc1_body(x_ref, w_ref, degp_ref, h0_ref, h1_ref, dis_ref):
    j = pl.program_id(0)
    deg = jnp.sum(degp_ref[...], axis=0) + 1.0
    dis = lax.rsqrt(deg)
    mm = jnp.dot(x_ref[...], w_ref[...], preferred_element_type=jnp.float32)
    row = j * RT + lax.broadcasted_iota(jnp.int32, (RT, 1), 0)
    hp = jnp.where(row < N, mm * dis[:, None], 0.0)
    h0_ref[...] = hp[:, :H]
    h1_ref[...] = hp[:, H:]
    dis_ref[...] = dis[:, None]


def _tc1(x, W1, degp):
    return pl.pallas_call(
        _tc1_body,
        grid=(NRT,),
        in_specs=[
            pl.BlockSpec((RT, D), lambda j: (j, 0)),
            pl.BlockSpec((D, D), lambda j: (0, 0)),
            pl.BlockSpec((NW, RT), lambda j: (0, j)),
        ],
        out_specs=[
            pl.BlockSpec((RT, H), lambda j: (j, 0)),
            pl.BlockSpec((RT, H), lambda j: (j, 0)),
            pl.BlockSpec((RT, 1), lambda j: (j, 0)),
        ],
        out_shape=[
            jax.ShapeDtypeStruct((NP, H), jnp.float32),
            jax.ShapeDtypeStruct((NP, H), jnp.float32),
            jax.ShapeDtypeStruct((NP, 1), jnp.float32),
        ],
    )(x, W1, degp)


def _bn_accum(j, z, scr):
    @pl.when(j == 0)
    def _():
        scr[...] = jnp.zeros_like(scr)

    @pl.when(j < NRT)
    def _():
        scr[...] += jnp.stack([jnp.sum(z, axis=0), jnp.sum(z * z, axis=0)])


def _bn_apply(z, scr, g_ref, be_ref):
    m = scr[0:1, :] * (1.0 / N)
    v = scr[1:2, :] * (1.0 / N) - m * m
    return jnp.maximum((z - m) * lax.rsqrt(v + 1e-5) * g_ref[...]
                       + be_ref[...], 0.0)


def _tcmid_body(a0_ref, a1_ref, dis_ref, g_ref, be_ref, w_ref,
                h0_ref, h1_ref, scr):
    j = pl.program_id(0)
    dis = dis_ref[...]
    z = jnp.concatenate([a0_ref[...] * dis, a1_ref[...] * dis], axis=1)
    _bn_accum(j, z, scr)

    @pl.when(j >= NRT)
    def _():
        y = _bn_apply(z, scr, g_ref, be_ref)
        mm = jnp.dot(y, w_ref[...], preferred_element_type=jnp.float32)
        row = (j - NRT) * RT + lax.broadcasted_iota(jnp.int32, (RT, 1), 0)
        hp = jnp.where(row < N, mm * dis, 0.0)
        h0_ref[...] = hp[:, :H]
        h1_ref[...] = hp[:, H:]


def _tcmid(a0, a1, dis2d, g, be, W2):
    blk = lambda j: (lax.rem(j, NRT), 0)
    out_blk = lambda j: (jnp.maximum(j - NRT, 0), 0)
    return pl.pallas_call(
        _tcmid_body,
        grid=(2 * NRT,),
        in_specs=[
            pl.BlockSpec((RT, H), blk),
            pl.BlockSpec((RT, H), blk),
            pl.BlockSpec((RT, 1), blk),
            pl.BlockSpec((1, D), lambda j: (0, 0)),
            pl.BlockSpec((1, D), lambda j: (0, 0)),
            pl.BlockSpec((D, D), lambda j: (0, 0)),
        ],
        out_specs=[
            pl.BlockSpec((RT, H), out_blk),
            pl.BlockSpec((RT, H), out_blk),
        ],
        out_shape=[
            jax.ShapeDtypeStruct((NP, H), jnp.float32),
            jax.ShapeDtypeStruct((NP, H), jnp.float32),
        ],
        scratch_shapes=[pltpu.VMEM((2, D), jnp.float32)],
    )(a0, a1, dis2d, g, be, W2)


def _tcfin_body(a0_ref, a1_ref, dis_ref, g_ref, be_ref, out_ref, scr):
    j = pl.program_id(0)
    dis = dis_ref[...]
    z = jnp.concatenate([a0_ref[...] * dis, a1_ref[...] * dis], axis=1)
    _bn_accum(j, z, scr)

    @pl.when(j >= NRT)
    def _():
        out_ref[...] = _bn_apply(z, scr, g_ref, be_ref)


def _tcfin(a0, a1, dis2d, g, be):
    blk = lambda j: (lax.rem(j, NRT), 0)
    out_blk = lambda j: (jnp.maximum(j - NRT, 0), 0)
    return pl.pallas_call(
        _tcfin_body,
        grid=(2 * NRT,),
        in_specs=[
            pl.BlockSpec((RT, H), blk),
            pl.BlockSpec((RT, H), blk),
            pl.BlockSpec((RT, 1), blk),
            pl.BlockSpec((1, D), lambda j: (0, 0)),
            pl.BlockSpec((1, D), lambda j: (0, 0)),
        ],
        out_specs=pl.BlockSpec((RT, D), out_blk),
        out_shape=jax.ShapeDtypeStruct((N, D), jnp.float32),
        scratch_shapes=[pltpu.VMEM((2, D), jnp.float32)],
    )(a0, a1, dis2d, g, be)


def kernel(x, support, W1, b1, g1, be1, W2, b2, g2, be2):
    del b1, b2
    src = support[0].astype(jnp.int32)
    dst = support[1].astype(jnp.int32)
    g1r = g1.reshape(1, D)
    be1r = be1.reshape(1, D)
    g2r = g2.reshape(1, D)
    be2r = be2.reshape(1, D)

    sd = jnp.stack([src.reshape(NCHUNK, K), dst.reshape(NCHUNK, K)], axis=1)

    degp = _deg_kernel(dst)
    h0, h1, dis2d = _tc1(x, W1, degp)
    a0, a1 = _agg_kernel(h0, h1, sd)
    h20, h21 = _tcmid(a0, a1, dis2d, g1r, be1r, W2)
    b0u, b1u = _agg_kernel(h20, h21, sd)
    return _tcfin(b0u, b1u, dis2d, g2r, be2r)


# speedup vs baseline: 2.5077x; 1.0179x over previous
import functools
import jax
import jax.numpy as jnp
from jax import lax
from jax.experimental import pallas as pl
from jax.experimental.pallas import tpu as pltpu
from jax.experimental.pallas import tpu_sc as plsc

N = 10000
E = 160000
D = 256
H = 128
NP = 10240
RT = 512
NRT = NP // RT
K = 128
NCHUNK = E // K
NTILES = 16
NW = 32
EPW = E // NW

_mesh = lambda: plsc.VectorSubcoreMesh(core_axis_name="c", subcore_axis_name="s")


def _deg_body(dst_hbm, out_hbm, dbuf, acc):
    c = lax.axis_index("c")
    s = lax.axis_index("s")
    w = s * 2 + c

    def zero(i, _):
        acc[pl.ds(i * 16, 16)] = jnp.zeros((16,), jnp.float32)
        return 0
    lax.fori_loop(0, NP // 16, zero, 0)

    dbuf[pl.ds(EPW - 16, 16)] = jnp.zeros((16,), jnp.int32)
    dbuf[pl.ds(EPW, 16)] = jnp.zeros((16,), jnp.int32)
    pltpu.sync_copy(dst_hbm.at[pl.ds(w * EPW, EPW)], dbuf.at[pl.ds(0, EPW)])

    ones = jnp.full((16,), 1.0, jnp.float32)

    def body(i, _):
        idx = dbuf[pl.ds(i * 16, 16)]
        plsc.addupdate_scatter(acc, [idx], ones)
        return 0
    lax.fori_loop(0, EPW // 16, body, 0)
    idx = dbuf[pl.ds((EPW // 16) * 16, 16)]
    mask = lax.broadcasted_iota(jnp.int32, (16,), 0) < (EPW % 16)
    plsc.addupdate_scatter(acc, [idx], ones, mask=mask)

    pltpu.sync_copy(acc, out_hbm.at[w])


@functools.partial(
    pl.kernel,
    out_type=jax.ShapeDtypeStruct((NW, NP), jnp.float32),
    mesh=_mesh(),
    scratch_types=[
        pltpu.VMEM((EPW + 16,), jnp.int32),
        pltpu.VMEM((NP,), jnp.float32),
    ],
    compiler_params=pltpu.CompilerParams(needs_layout_passes=False),
)
def _deg_kernel(dst_hbm, out_hbm, dbuf, acc):
    _deg_body(dst_hbm, out_hbm, dbuf, acc)


K = 128
NCHUNK = E // K


def _agg_init(tbl, acc, s, rows_v):
    del rows_v
    base = s * (NP // NTILES)
    pltpu.sync_copy(tbl.at[pl.ds(base, NP // NTILES)],
                    acc.at[pl.ds(base, NP // NTILES)])


def _agg_edges(tbl, acc, s, sd_hbm, sdA, sdB, rowsA, rowsB,
               ixa, ixb, ga, gb):
    n = 78 + jnp.where(s < NCHUNK - 78 * NTILES, 1, 0)

    def load(i, buf, sem):
        pltpu.async_copy(sd_hbm.at[s + NTILES * i], buf, sem)

    def wait_ix(buf, sem):
        pltpu.make_async_copy(sd_hbm.at[0], buf, sem).wait()

    def gather(idx, rows, sem):
        pltpu.async_copy(tbl.at[idx.at[0]], rows, sem)

    def wait_g(rows, sem):
        pltpu.make_async_copy(tbl.at[sdA.at[0]], rows, sem).wait()

    def scatter(idx, rows):
        pltpu.sync_copy(rows, acc.at[idx.at[1]], add=True)

    load(0, sdA, ixa)
    load(1, sdB, ixb)
    wait_ix(sdA, ixa)
    gather(sdA, rowsA, ga)

    def body(j, _):
        wait_g(rowsA, ga)
        wait_ix(sdB, ixb)
        gather(sdB, rowsB, gb)
        scatter(sdA, rowsA)

        @pl.when(2 * j + 2 < n)
        def _():
            load(2 * j + 2, sdA, ixa)

        wait_g(rowsB, gb)

        @pl.when(2 * j + 2 < n)
        def _():
            wait_ix(sdA, ixa)
            gather(sdA, rowsA, ga)
        scatter(sdB, rowsB)

        @pl.when(2 * j + 3 < n)
        def _():
            load(2 * j + 3, sdB, ixb)
        return 0
    lax.fori_loop(0, n // 2, body, 0)

    @pl.when(n % 2 == 1)
    def _():
        wait_g(rowsA, ga)
        scatter(sdA, rowsA)


def _agg_writeout(out, acc, s, rows_v):
    del rows_v
    base = s * (NP // NTILES)
    pltpu.sync_copy(acc.at[pl.ds(base, NP // NTILES)],
                    out.at[pl.ds(base, NP // NTILES)])


@functools.partial(
    pl.kernel,
    out_type=(
        jax.ShapeDtypeStruct((NP, H), jnp.float32),
        jax.ShapeDtypeStruct((NP, H), jnp.float32),
    ),
    mesh=_mesh(),
    scratch_types=[
        pltpu.VMEM((2, K), jnp.int32),
        pltpu.VMEM((2, K), jnp.int32),
        pltpu.VMEM((K, H), jnp.float32),
        pltpu.VMEM((K, H), jnp.float32),
        pltpu.VMEM_SHARED((NP, H), jnp.float32),
        pltpu.SemaphoreType.DMA,
        pltpu.SemaphoreType.DMA,
        pltpu.SemaphoreType.DMA,
        pltpu.SemaphoreType.DMA,
    ],
    compiler_params=pltpu.CompilerParams(needs_layout_passes=False),
)
def _agg_kernel(h0, h1, sd_hbm, o0, o1, sdA, sdB, rowsA, rowsB, acc,
                ixa, ixb, ga, gb):
    c = lax.axis_index("c")
    s = lax.axis_index("s")

    @pl.when(c == 0)
    def _():
        _agg_init(h0, acc, s, rowsA)

    @pl.when(c == 1)
    def _():
        _agg_init(h1, acc, s, rowsA)

    plsc.subcore_barrier()

    @pl.when(c == 0)
    def _():
        _agg_edges(h0, acc, s, sd_hbm, sdA, sdB, rowsA, rowsB,
                   ixa, ixb, ga, gb)

    @pl.when(c == 1)
    def _():
        _agg_edges(h1, acc, s, sd_hbm, sdA, sdB, rowsA, rowsB,
                   ixa, ixb, ga, gb)

    plsc.subcore_barrier()

    @pl.when(c == 0)
    def _():
        _agg_writeout(o0, acc, s, rowsA)

    @pl.when(c == 1)
    def _():
        _agg_writeout(o1, acc, s, rowsA)


def _tc1_body(x_ref, w_ref, degp_ref, h0_ref, h1_ref, dis_ref):
    j = pl.program_id(0)
    deg = jnp.sum(degp_ref[...], axis=0) + 1.0
    dis = lax.rsqrt(deg)
    mm = jnp.dot(x_ref[...], w_ref[...], preferred_element_type=jnp.float32)
    row = j * RT + lax.broadcasted_iota(jnp.int32, (RT, 1), 0)
    hp = jnp.where(row < N, mm * dis[:, None], 0.0)
    h0_ref[...] = hp[:, :H]
    h1_ref[...] = hp[:, H:]
    dis_ref[...] = dis[:, None]


def _tc1(x, W1, degp):
    return pl.pallas_call(
        _tc1_body,
        grid=(NRT,),
        in_specs=[
            pl.BlockSpec((RT, D), lambda j: (j, 0)),
            pl.BlockSpec((D, D), lambda j: (0, 0)),
            pl.BlockSpec((NW, RT), lambda j: (0, j)),
        ],
        out_specs=[
            pl.BlockSpec((RT, H), lambda j: (j, 0)),
            pl.BlockSpec((RT, H), lambda j: (j, 0)),
            pl.BlockSpec((RT, 1), lambda j: (j, 0)),
        ],
        out_shape=[
            jax.ShapeDtypeStruct((NP, H), jnp.float32),
            jax.ShapeDtypeStruct((NP, H), jnp.float32),
            jax.ShapeDtypeStruct((NP, 1), jnp.float32),
        ],
    )(x, W1, degp)


def _bn_accum(j, z, scr):
    @pl.when(j == 0)
    def _():
        scr[...] = jnp.zeros_like(scr)

    @pl.when(j < NRT)
    def _():
        scr[...] += jnp.stack([jnp.sum(z, axis=0), jnp.sum(z * z, axis=0)])


def _bn_apply(z, scr, g_ref, be_ref):
    m = scr[0:1, :] * (1.0 / N)
    v = scr[1:2, :] * (1.0 / N) - m * m
    return jnp.maximum((z - m) * lax.rsqrt(v + 1e-5) * g_ref[...]
                       + be_ref[...], 0.0)


def _tcmid_body(a0_ref, a1_ref, dis_ref, g_ref, be_ref, w_ref,
                h0_ref, h1_ref, scr):
    j = pl.program_id(0)
    dis = dis_ref[...]
    z = jnp.concatenate([a0_ref[...] * dis, a1_ref[...] * dis], axis=1)
    _bn_accum(j, z, scr)

    @pl.when(j >= NRT)
    def _():
        y = _bn_apply(z, scr, g_ref, be_ref)
        mm = jnp.dot(y, w_ref[...], preferred_element_type=jnp.float32)
        row = (j - NRT) * RT + lax.broadcasted_iota(jnp.int32, (RT, 1), 0)
        hp = jnp.where(row < N, mm * dis, 0.0)
        h0_ref[...] = hp[:, :H]
        h1_ref[...] = hp[:, H:]


def _tcmid(a0, a1, dis2d, g, be, W2):
    blk = lambda j: (lax.rem(j, NRT), 0)
    out_blk = lambda j: (jnp.maximum(j - NRT, 0), 0)
    return pl.pallas_call(
        _tcmid_body,
        grid=(2 * NRT,),
        in_specs=[
            pl.BlockSpec((RT, H), blk),
            pl.BlockSpec((RT, H), blk),
            pl.BlockSpec((RT, 1), blk),
            pl.BlockSpec((1, D), lambda j: (0, 0)),
            pl.BlockSpec((1, D), lambda j: (0, 0)),
            pl.BlockSpec((D, D), lambda j: (0, 0)),
        ],
        out_specs=[
            pl.BlockSpec((RT, H), out_blk),
            pl.BlockSpec((RT, H), out_blk),
        ],
        out_shape=[
            jax.ShapeDtypeStruct((NP, H), jnp.float32),
            jax.ShapeDtypeStruct((NP, H), jnp.float32),
        ],
        scratch_shapes=[pltpu.VMEM((2, D), jnp.float32)],
    )(a0, a1, dis2d, g, be, W2)


def _tcfin_body(a0_ref, a1_ref, dis_ref, g_ref, be_ref, out_ref, scr):
    j = pl.program_id(0)
    dis = dis_ref[...]
    z = jnp.concatenate([a0_ref[...] * dis, a1_ref[...] * dis], axis=1)
    _bn_accum(j, z, scr)

    @pl.when(j >= NRT)
    def _():
        out_ref[...] = _bn_apply(z, scr, g_ref, be_ref)


def _tcfin(a0, a1, dis2d, g, be):
    blk = lambda j: (lax.rem(j, NRT), 0)
    out_blk = lambda j: (jnp.maximum(j - NRT, 0), 0)
    return pl.pallas_call(
        _tcfin_body,
        grid=(2 * NRT,),
        in_specs=[
            pl.BlockSpec((RT, H), blk),
            pl.BlockSpec((RT, H), blk),
            pl.BlockSpec((RT, 1), blk),
            pl.BlockSpec((1, D), lambda j: (0, 0)),
            pl.BlockSpec((1, D), lambda j: (0, 0)),
        ],
        out_specs=pl.BlockSpec((RT, D), out_blk),
        out_shape=jax.ShapeDtypeStruct((N, D), jnp.float32),
        scratch_shapes=[pltpu.VMEM((2, D), jnp.float32)],
    )(a0, a1, dis2d, g, be)


def kernel(x, support, W1, b1, g1, be1, W2, b2, g2, be2):
    del b1, b2
    src = support[0].astype(jnp.int32)
    dst = support[1].astype(jnp.int32)
    g1r = g1.reshape(1, D)
    be1r = be1.reshape(1, D)
    g2r = g2.reshape(1, D)
    be2r = be2.reshape(1, D)

    sd = jnp.stack([src.reshape(NCHUNK, K), dst.reshape(NCHUNK, K)], axis=1)

    degp = _deg_kernel(dst)
    h0, h1, dis2d = _tc1(x, W1, degp)
    a0, a1 = _agg_kernel(h0, h1, sd)
    h20, h21 = _tcmid(a0, a1, dis2d, g1r, be1r, W2)
    b0u, b1u = _agg_kernel(h20, h21, sd)
    return _tcfin(b0u, b1u, dis2d, g2r, be2r)


# speedup vs baseline: 2.5104x; 1.0011x over previous
import functools
import jax
import jax.numpy as jnp
from jax import lax
from jax.experimental import pallas as pl
from jax.experimental.pallas import tpu as pltpu
from jax.experimental.pallas import tpu_sc as plsc

N = 10000
E = 160000
D = 256
H = 128
NP = 10240
RT = 512
NRT = NP // RT
K = 128
NCHUNK = E // K
NTILES = 16
NW = 32
EPW = E // NW

_mesh = lambda: plsc.VectorSubcoreMesh(core_axis_name="c", subcore_axis_name="s")


def _deg_body(dst_hbm, out_hbm, dbuf, acc):
    c = lax.axis_index("c")
    s = lax.axis_index("s")
    w = s * 2 + c

    def zero(i, _):
        acc[pl.ds(i * 16, 16)] = jnp.zeros((16,), jnp.float32)
        return 0
    lax.fori_loop(0, NP // 16, zero, 0)

    dbuf[pl.ds(EPW - 16, 16)] = jnp.zeros((16,), jnp.int32)
    dbuf[pl.ds(EPW, 16)] = jnp.zeros((16,), jnp.int32)
    pltpu.sync_copy(dst_hbm.at[pl.ds(w * EPW, EPW)], dbuf.at[pl.ds(0, EPW)])

    ones = jnp.full((16,), 1.0, jnp.float32)

    def body(i, _):
        idx = dbuf[pl.ds(i * 16, 16)]
        plsc.addupdate_scatter(acc, [idx], ones)
        return 0
    lax.fori_loop(0, EPW // 16, body, 0)
    idx = dbuf[pl.ds((EPW // 16) * 16, 16)]
    mask = lax.broadcasted_iota(jnp.int32, (16,), 0) < (EPW % 16)
    plsc.addupdate_scatter(acc, [idx], ones, mask=mask)

    pltpu.sync_copy(acc, out_hbm.at[w])


@functools.partial(
    pl.kernel,
    out_type=jax.ShapeDtypeStruct((NW, NP), jnp.float32),
    mesh=_mesh(),
    scratch_types=[
        pltpu.VMEM((EPW + 16,), jnp.int32),
        pltpu.VMEM((NP,), jnp.float32),
    ],
    compiler_params=pltpu.CompilerParams(needs_layout_passes=False),
)
def _deg_kernel(dst_hbm, out_hbm, dbuf, acc):
    _deg_body(dst_hbm, out_hbm, dbuf, acc)


K = 128
NCHUNK = E // K


def _agg_init(tbl, acc, s):
    base = s * (NP // NTILES)
    pltpu.sync_copy(tbl.at[pl.ds(base, NP // NTILES)],
                    acc.at[pl.ds(base, NP // NTILES)])


def _agg_edges(tbl, acc, s, sd_hbm, sdA, sdB, rowsA, rowsB,
               ixa, ixb, ga, gb):
    n = 78 + jnp.where(s < NCHUNK - 78 * NTILES, 1, 0)

    def load(i, buf, sem):
        pltpu.async_copy(sd_hbm.at[s + NTILES * i], buf, sem)

    def wait_ix(buf, sem):
        pltpu.make_async_copy(sd_hbm.at[0], buf, sem).wait()

    def gather(idx, rows, sem):
        pltpu.async_copy(tbl.at[idx.at[0]], rows, sem)

    def wait_g(rows, sem):
        pltpu.make_async_copy(tbl.at[sdA.at[0]], rows, sem).wait()

    def scatter(idx, rows):
        pltpu.sync_copy(rows, acc.at[idx.at[1]], add=True)

    load(0, sdA, ixa)
    load(1, sdB, ixb)
    wait_ix(sdA, ixa)
    gather(sdA, rowsA, ga)

    def body(j, _):
        wait_g(rowsA, ga)
        wait_ix(sdB, ixb)
        gather(sdB, rowsB, gb)
        scatter(sdA, rowsA)

        @pl.when(2 * j + 2 < n)
        def _():
            load(2 * j + 2, sdA, ixa)

        wait_g(rowsB, gb)

        @pl.when(2 * j + 2 < n)
        def _():
            wait_ix(sdA, ixa)
            gather(sdA, rowsA, ga)
        scatter(sdB, rowsB)

        @pl.when(2 * j + 3 < n)
        def _():
            load(2 * j + 3, sdB, ixb)
        return 0
    lax.fori_loop(0, n // 2, body, 0)

    @pl.when(n % 2 == 1)
    def _():
        wait_g(rowsA, ga)
        scatter(sdA, rowsA)


def _agg_writeout(out, acc, s):
    base = s * (NP // NTILES)
    pltpu.sync_copy(acc.at[pl.ds(base, NP // NTILES)],
                    out.at[pl.ds(base, NP // NTILES)])


@functools.partial(
    pl.kernel,
    out_type=(
        jax.ShapeDtypeStruct((NP, H), jnp.float32),
        jax.ShapeDtypeStruct((NP, H), jnp.float32),
    ),
    mesh=_mesh(),
    scratch_types=[
        pltpu.VMEM((2, K), jnp.int32),
        pltpu.VMEM((2, K), jnp.int32),
        pltpu.VMEM((K, H), jnp.float32),
        pltpu.VMEM((K, H), jnp.float32),
        pltpu.VMEM_SHARED((NP, H), jnp.float32),
        pltpu.SemaphoreType.DMA,
        pltpu.SemaphoreType.DMA,
        pltpu.SemaphoreType.DMA,
        pltpu.SemaphoreType.DMA,
    ],
    compiler_params=pltpu.CompilerParams(needs_layout_passes=False),
)
def _agg_kernel(h0, h1, sd_hbm, o0, o1, sdA, sdB, rowsA, rowsB, acc,
                ixa, ixb, ga, gb):
    c = lax.axis_index("c")
    s = lax.axis_index("s")

    @pl.when(c == 0)
    def _():
        _agg_init(h0, acc, s)

    @pl.when(c == 1)
    def _():
        _agg_init(h1, acc, s)

    plsc.subcore_barrier()

    @pl.when(c == 0)
    def _():
        _agg_edges(h0, acc, s, sd_hbm, sdA, sdB, rowsA, rowsB,
                   ixa, ixb, ga, gb)

    @pl.when(c == 1)
    def _():
        _agg_edges(h1, acc, s, sd_hbm, sdA, sdB, rowsA, rowsB,
                   ixa, ixb, ga, gb)

    plsc.subcore_barrier()

    @pl.when(c == 0)
    def _():
        _agg_writeout(o0, acc, s)

    @pl.when(c == 1)
    def _():
        _agg_writeout(o1, acc, s)


def _tc1_body(x_ref, w_ref, degp_ref, h0_ref, h1_ref, dis_ref):
    j = pl.program_id(0)
    deg = jnp.sum(degp_ref[...], axis=0) + 1.0
    dis = lax.rsqrt(deg)
    mm = jnp.dot(x_ref[...], w_ref[...], preferred_element_type=jnp.float32)
    row = j * RT + lax.broadcasted_iota(jnp.int32, (RT, 1), 0)
    hp = jnp.where(row < N, mm * dis[:, None], 0.0)
    h0_ref[...] = hp[:, :H]
    h1_ref[...] = hp[:, H:]
    dis_ref[...] = dis[:, None]


def _tc1(x, W1, degp):
    return pl.pallas_call(
        _tc1_body,
        grid=(NRT,),
        in_specs=[
            pl.BlockSpec((RT, D), lambda j: (j, 0)),
            pl.BlockSpec((D, D), lambda j: (0, 0)),
            pl.BlockSpec((NW, RT), lambda j: (0, j)),
        ],
        out_specs=[
            pl.BlockSpec((RT, H), lambda j: (j, 0)),
            pl.BlockSpec((RT, H), lambda j: (j, 0)),
            pl.BlockSpec((RT, 1), lambda j: (j, 0)),
        ],
        out_shape=[
            jax.ShapeDtypeStruct((NP, H), jnp.float32),
            jax.ShapeDtypeStruct((NP, H), jnp.float32),
            jax.ShapeDtypeStruct((NP, 1), jnp.float32),
        ],
    )(x, W1, degp)


def _bn_accum(j, z, scr):
    @pl.when(j == 0)
    def _():
        scr[...] = jnp.zeros_like(scr)

    @pl.when(j < NRT)
    def _():
        scr[...] += jnp.stack([jnp.sum(z, axis=0), jnp.sum(z * z, axis=0)])


def _bn_apply(z, scr, g_ref, be_ref):
    m = scr[0:1, :] * (1.0 / N)
    v = scr[1:2, :] * (1.0 / N) - m * m
    return jnp.maximum((z - m) * lax.rsqrt(v + 1e-5) * g_ref[...]
                       + be_ref[...], 0.0)


def _tcmid_body(a0_ref, a1_ref, dis_ref, g_ref, be_ref, w_ref,
                h0_ref, h1_ref, scr):
    j = pl.program_id(0)
    dis = dis_ref[...]
    z = jnp.concatenate([a0_ref[...] * dis, a1_ref[...] * dis], axis=1)
    _bn_accum(j, z, scr)

    @pl.when(j >= NRT)
    def _():
        y = _bn_apply(z, scr, g_ref, be_ref)
        mm = jnp.dot(y, w_ref[...], preferred_element_type=jnp.float32)
        row = (j - NRT) * RT + lax.broadcasted_iota(jnp.int32, (RT, 1), 0)
        hp = jnp.where(row < N, mm * dis, 0.0)
        h0_ref[...] = hp[:, :H]
        h1_ref[...] = hp[:, H:]


def _tcmid(a0, a1, dis2d, g, be, W2):
    blk = lambda j: (lax.rem(j, NRT), 0)
    out_blk = lambda j: (jnp.maximum(j - NRT, 0), 0)
    return pl.pallas_call(
        _tcmid_body,
        grid=(2 * NRT,),
        in_specs=[
            pl.BlockSpec((RT, H), blk),
            pl.BlockSpec((RT, H), blk),
            pl.BlockSpec((RT, 1), blk),
            pl.BlockSpec((1, D), lambda j: (0, 0)),
            pl.BlockSpec((1, D), lambda j: (0, 0)),
            pl.BlockSpec((D, D), lambda j: (0, 0)),
        ],
        out_specs=[
            pl.BlockSpec((RT, H), out_blk),
            pl.BlockSpec((RT, H), out_blk),
        ],
        out_shape=[
            jax.ShapeDtypeStruct((NP, H), jnp.float32),
            jax.ShapeDtypeStruct((NP, H), jnp.float32),
        ],
        scratch_shapes=[pltpu.VMEM((2, D), jnp.float32)],
    )(a0, a1, dis2d, g, be, W2)


def _tcfin_body(a0_ref, a1_ref, dis_ref, g_ref, be_ref, out_ref, scr):
    j = pl.program_id(0)
    dis = dis_ref[...]
    z = jnp.concatenate([a0_ref[...] * dis, a1_ref[...] * dis], axis=1)
    _bn_accum(j, z, scr)

    @pl.when(j >= NRT)
    def _():
        out_ref[...] = _bn_apply(z, scr, g_ref, be_ref)


def _tcfin(a0, a1, dis2d, g, be):
    blk = lambda j: (lax.rem(j, NRT), 0)
    out_blk = lambda j: (jnp.maximum(j - NRT, 0), 0)
    return pl.pallas_call(
        _tcfin_body,
        grid=(2 * NRT,),
        in_specs=[
            pl.BlockSpec((RT, H), blk),
            pl.BlockSpec((RT, H), blk),
            pl.BlockSpec((RT, 1), blk),
            pl.BlockSpec((1, D), lambda j: (0, 0)),
            pl.BlockSpec((1, D), lambda j: (0, 0)),
        ],
        out_specs=pl.BlockSpec((RT, D), out_blk),
        out_shape=jax.ShapeDtypeStruct((N, D), jnp.float32),
        scratch_shapes=[pltpu.VMEM((2, D), jnp.float32)],
    )(a0, a1, dis2d, g, be)


def kernel(x, support, W1, b1, g1, be1, W2, b2, g2, be2):
    del b1, b2
    src = support[0].astype(jnp.int32)
    dst = support[1].astype(jnp.int32)
    g1r = g1.reshape(1, D)
    be1r = be1.reshape(1, D)
    g2r = g2.reshape(1, D)
    be2r = be2.reshape(1, D)

    sd = jnp.stack([src.reshape(NCHUNK, K), dst.reshape(NCHUNK, K)], axis=1)

    degp = _deg_kernel(dst)
    h0, h1, dis2d = _tc1(x, W1, degp)
    a0, a1 = _agg_kernel(h0, h1, sd)
    h20, h21 = _tcmid(a0, a1, dis2d, g1r, be1r, W2)
    b0u, b1u = _agg_kernel(h20, h21, sd)
    return _tcfin(b0u, b1u, dis2d, g2r, be2r)
